# Initial kernel scaffold; baseline (speedup 1.0000x reference)
import jax
import jax.numpy as jnp
from jax.experimental import pallas as pl


def kernel(x, relative_position_bias_table, rpe_index):
    raise NotImplementedError("write your pallas kernel here")


# speedup vs baseline: 12.2492x; 12.2492x over previous
import functools

import jax
import jax.numpy as jnp
from jax import lax
from jax.experimental import pallas as pl
from jax.experimental.pallas import tpu as pltpu
from jax.experimental.pallas import tpu_sc as plsc


def kernel(x, relative_position_bias_table, rpe_index):
    B, H, N, N2 = x.shape
    V = relative_position_bias_table.shape[0]
    L = 16

    tbl_flat = jnp.reshape(
        jnp.transpose(relative_position_bias_table), (-1,))
    idx = rpe_index.astype(jnp.int32)

    info = plsc.get_sparse_core_info()
    nw = info.num_cores * info.num_subcores
    rows_per_w = N // nw

    mesh = plsc.VectorSubcoreMesh(core_axis_name="c", subcore_axis_name="s")

    @functools.partial(
        pl.kernel,
        mesh=mesh,
        out_type=jax.ShapeDtypeStruct((B, H, N, N2), jnp.float32),
        compiler_params=pltpu.CompilerParams(needs_layout_passes=False),
        scratch_types=[
            pltpu.VMEM((H * V,), jnp.float32),
            pltpu.VMEM((N2,), jnp.int32),
            pltpu.VMEM((B, H, N2), jnp.float32)
        ],
    )
    def run(x_hbm, tbl_hbm, idx_hbm, out_hbm, tbl_v, idx_v, x_v):
        wid = lax.axis_index("s") * info.num_cores + lax.axis_index("c")
        base = wid * rows_per_w
        pltpu.sync_copy(tbl_hbm, tbl_v)

        def row_body(r, carry):
            i = base + r
            pltpu.sync_copy(idx_hbm.at[i], idx_v)
            pltpu.sync_copy(x_hbm.at[:, :, i], x_v)

            def v_body(v, c):
                start = pl.multiple_of(v * L, L)
                iv = idx_v[pl.ds(start, L)]
                for h in range(H):
                    bias = plsc.load_gather(tbl_v, [iv + h * V])
                    for b in range(B):
                        seg = x_v[b, h, pl.ds(start, L)]
                        x_v[b, h, pl.ds(start, L)] = seg + bias
                return c

            lax.fori_loop(0, N2 // L, v_body, 0)
            pltpu.sync_copy(x_v, out_hbm.at[:, :, i])
            return carry

        lax.fori_loop(0, rows_per_w, row_body, 0)

    return run(x, tbl_flat, idx)


# speedup vs baseline: 16.8677x; 1.3770x over previous
import functools

import jax
import jax.numpy as jnp
from jax import lax
from jax.experimental import pallas as pl
from jax.experimental.pallas import tpu as pltpu
from jax.experimental.pallas import tpu_sc as plsc


def kernel(x, relative_position_bias_table, rpe_index):
    B, H, N, N2 = x.shape
    V = relative_position_bias_table.shape[0]
    L = 16
    C = 512

    tbl_flat = jnp.reshape(
        jnp.transpose(relative_position_bias_table), (-1,))
    idx = rpe_index.astype(jnp.int32)

    info = plsc.get_sparse_core_info()
    nw = info.num_cores * info.num_subcores
    rows_per_w = N // nw

    mesh = plsc.VectorSubcoreMesh(core_axis_name="c", subcore_axis_name="s")

    @functools.partial(
        pl.kernel,
        mesh=mesh,
        out_type=jax.ShapeDtypeStruct((B, H, N, N2), jnp.float32),
        compiler_params=pltpu.CompilerParams(needs_layout_passes=False),
        scratch_types=[
            pltpu.VMEM((H * V,), jnp.float32),
            pltpu.VMEM((C,), jnp.int32),
            pltpu.VMEM((C,), jnp.int32),
            pltpu.VMEM((B, H, C), jnp.float32),
            pltpu.VMEM((B, H, C), jnp.float32),
            pltpu.VMEM((B, H, C), jnp.float32),
            pltpu.VMEM((B, H, C), jnp.float32),
            pltpu.SemaphoreType.DMA,
            pltpu.SemaphoreType.DMA,
            pltpu.SemaphoreType.DMA,
            pltpu.SemaphoreType.DMA,
            pltpu.SemaphoreType.DMA,
            pltpu.SemaphoreType.DMA,
        ],
    )
    def run(x_hbm, tbl_hbm, idx_hbm, out_hbm,
            tbl_v, idx0, idx1, xin0, xin1, xo0, xo1,
            si0, si1, sx0, sx1, so0, so1):
        wid = lax.axis_index("s") * info.num_cores + lax.axis_index("c")
        base = wid * rows_per_w
        pltpu.sync_copy(tbl_hbm, tbl_v)

        idxb = (idx0, idx1)
        xinb = (xin0, xin1)
        xob = (xo0, xo1)
        sib = (si0, si1)
        sxb = (sx0, sx1)
        sob = (so0, so1)

        def issue_in(row, k):
            j0 = k * C
            pltpu.async_copy(
                idx_hbm.at[row, pl.ds(j0, C)], idxb[k], sib[k])
            pltpu.async_copy(
                x_hbm.at[:, :, row, pl.ds(j0, C)], xinb[k], sxb[k])

        def substep(r, k):
            row = base + r
            j0 = k * C
            pltpu.make_async_copy(
                idx_hbm.at[row, pl.ds(j0, C)], idxb[k], sib[k]).wait()
            pltpu.make_async_copy(
                x_hbm.at[:, :, row, pl.ds(j0, C)], xinb[k], sxb[k]).wait()

            @pl.when(r >= 1)
            def _():
                pltpu.make_async_copy(
                    xob[k], out_hbm.at[:, :, row - 1, pl.ds(j0, C)],
                    sob[k]).wait()

            def v_body(v, c):
                start = pl.multiple_of(v * L, L)
                iv = idxb[k][pl.ds(start, L)]
                for h in range(H):
                    bias = plsc.load_gather(tbl_v, [iv + h * V])
                    for b in range(B):
                        xob[k][b, h, pl.ds(start, L)] = (
                            xinb[k][b, h, pl.ds(start, L)] + bias)
                return c

            lax.fori_loop(0, C // L, v_body, 0)

            pltpu.async_copy(
                xob[k], out_hbm.at[:, :, row, pl.ds(j0, C)], sob[k])

            @pl.when(r + 1 < rows_per_w)
            def _():
                issue_in(row + 1, k)

        issue_in(base, 0)
        issue_in(base, 1)

        def row_body(r, carry):
            substep(r, 0)
            substep(r, 1)
            return carry

        lax.fori_loop(0, rows_per_w, row_body, 0)

        last = base + rows_per_w - 1
        for k in range(2):
            pltpu.make_async_copy(
                xob[k], out_hbm.at[:, :, last, pl.ds(k * C, C)],
                sob[k]).wait()

    return run(x, tbl_flat, idx)


# speedup vs baseline: 34.8245x; 2.0646x over previous
import functools

import jax
import jax.numpy as jnp
from jax import lax
from jax.experimental import pallas as pl
from jax.experimental.pallas import tpu as pltpu
from jax.experimental.pallas import tpu_sc as plsc


def kernel(x, relative_position_bias_table, rpe_index):
    B, H, N, N2 = x.shape
    V = relative_position_bias_table.shape[0]
    L = 16
    C = 512

    tbl_flat = jnp.reshape(
        jnp.transpose(relative_position_bias_table), (-1,))
    idx = rpe_index.astype(jnp.int32)

    info = plsc.get_sparse_core_info()
    nw = info.num_cores * info.num_subcores
    rows_per_w = N // nw

    mesh = plsc.VectorSubcoreMesh(core_axis_name="c", subcore_axis_name="s")

    @functools.partial(
        pl.kernel,
        mesh=mesh,
        out_type=jax.ShapeDtypeStruct((B, H, N, N2), jnp.float32),
        compiler_params=pltpu.CompilerParams(needs_layout_passes=False),
        scratch_types=[
            pltpu.VMEM((H * V,), jnp.float32),
            pltpu.VMEM((C,), jnp.int32),
            pltpu.VMEM((C,), jnp.int32),
            pltpu.VMEM((B, H, C), jnp.float32),
            pltpu.VMEM((B, H, C), jnp.float32),
            pltpu.VMEM((B, H, C), jnp.float32),
            pltpu.VMEM((B, H, C), jnp.float32),
            pltpu.SemaphoreType.DMA,
            pltpu.SemaphoreType.DMA,
            pltpu.SemaphoreType.DMA,
            pltpu.SemaphoreType.DMA,
            pltpu.SemaphoreType.DMA,
            pltpu.SemaphoreType.DMA,
        ],
    )
    def run(x_hbm, tbl_hbm, idx_hbm, out_hbm,
            tbl_v, idx0, idx1, xin0, xin1, xo0, xo1,
            si0, si1, sx0, sx1, so0, so1):
        wid = lax.axis_index("s") * info.num_cores + lax.axis_index("c")
        base = wid * rows_per_w
        pltpu.sync_copy(tbl_hbm, tbl_v)

        idxb = (idx0, idx1)
        xinb = (xin0, xin1)
        xob = (xo0, xo1)
        sib = (si0, si1)
        sxb = (sx0, sx1)
        sob = (so0, so1)

        def issue_in(row, k):
            j0 = k * C
            pltpu.async_copy(
                idx_hbm.at[row, pl.ds(j0, C)], idxb[k], sib[k])
            pltpu.async_copy(
                x_hbm.at[:, :, row, pl.ds(j0, C)], xinb[k], sxb[k])

        def substep(r, k):
            row = base + r
            j0 = k * C
            pltpu.make_async_copy(
                idx_hbm.at[row, pl.ds(j0, C)], idxb[k], sib[k]).wait()
            pltpu.make_async_copy(
                x_hbm.at[:, :, row, pl.ds(j0, C)], xinb[k], sxb[k]).wait()

            @pl.when(r >= 1)
            def _():
                pltpu.make_async_copy(
                    xob[k], out_hbm.at[:, :, row - 1, pl.ds(j0, C)],
                    sob[k]).wait()

            @plsc.parallel_loop(0, C // L, 1, unroll=2)
            def _(v):
                start = pl.multiple_of(v * L, L)
                iv = idxb[k][pl.ds(start, L)]
                for h in range(H):
                    bias = plsc.load_gather(tbl_v, [iv + h * V])
                    for b in range(B):
                        xob[k][b, h, pl.ds(start, L)] = (
                            xinb[k][b, h, pl.ds(start, L)] + bias)

            pltpu.async_copy(
                xob[k], out_hbm.at[:, :, row, pl.ds(j0, C)], sob[k])

            @pl.when(r + 1 < rows_per_w)
            def _():
                issue_in(row + 1, k)

        issue_in(base, 0)
        issue_in(base, 1)

        def row_body(r, carry):
            substep(r, 0)
            substep(r, 1)
            return carry

        lax.fori_loop(0, rows_per_w, row_body, 0)

        last = base + rows_per_w - 1
        for k in range(2):
            pltpu.make_async_copy(
                xob[k], out_hbm.at[:, :, last, pl.ds(k * C, C)],
                sob[k]).wait()

    return run(x, tbl_flat, idx)
